# baseline (speedup 1.0000x reference)
import functools

import jax
import jax.numpy as jnp
from jax import lax
from jax.experimental import pallas as pl
from jax.experimental.pallas import tpu as pltpu
from jax.experimental.pallas import tpu_sc as plsc

NC = 2
NS = 16
NW = NC * NS

L = 16


def _sc_body(num_images, C, H, W, B, P,
             ii_hbm, grid_hbm, feat_hbm, out_hbm,
             ii_v, grid_v, feat_v, idx_v, w_v, out_v, sem):
    HW = H * W
    CHW = C * HW
    n_b = B // NW
    n_pc = P // L

    wid = lax.axis_index("s") * NC + lax.axis_index("c")
    base = wid * n_b

    pltpu.sync_copy(ii_hbm.at[pl.ds(base, n_b)], ii_v)

    def per_query(i, _):
        b = base + i
        img = plsc.load_gather(ii_v, [jnp.full((L,), i, jnp.int32)])[0]
        pltpu.async_copy(feat_hbm.at[pl.ds(img * CHW, CHW)], feat_v, sem).wait()
        pltpu.sync_copy(grid_hbm.at[pl.ds(b, 1)], grid_v)

        def weights(pc, _):
            off = pc * L
            x = grid_v[0, 0, pl.ds(off, L)]
            y = grid_v[0, 1, pl.ds(off, L)]
            ix = jnp.clip(x * float(W) - 0.5, -1.0, float(W))
            iy = jnp.clip(y * float(H) - 0.5, -1.0, float(H))
            ix0 = (ix + float(W)).astype(jnp.int32) - W
            iy0 = (iy + float(H)).astype(jnp.int32) - H
            wx1 = ix - ix0.astype(jnp.float32)
            wy1 = iy - iy0.astype(jnp.float32)
            wx0 = 1.0 - wx1
            wy0 = 1.0 - wy1
            ix0c = jnp.clip(ix0, 0, W - 1)
            ix1c = jnp.clip(ix0 + 1, 0, W - 1)
            iy0c = jnp.clip(iy0, 0, H - 1) * W
            iy1c = jnp.clip(iy0 + 1, 0, H - 1) * W
            idx_v[0, pl.ds(off, L)] = iy0c + ix0c
            idx_v[1, pl.ds(off, L)] = iy0c + ix1c
            idx_v[2, pl.ds(off, L)] = iy1c + ix0c
            idx_v[3, pl.ds(off, L)] = iy1c + ix1c
            w_v[0, pl.ds(off, L)] = wy0 * wx0
            w_v[1, pl.ds(off, L)] = wy0 * wx1
            w_v[2, pl.ds(off, L)] = wy1 * wx0
            w_v[3, pl.ds(off, L)] = wy1 * wx1
            return 0

        lax.fori_loop(0, n_pc, weights, 0)

        def per_chunk(pc, _):
            off = pc * L
            i00 = idx_v[0, pl.ds(off, L)]
            i01 = idx_v[1, pl.ds(off, L)]
            i10 = idx_v[2, pl.ds(off, L)]
            i11 = idx_v[3, pl.ds(off, L)]
            w00 = w_v[0, pl.ds(off, L)]
            w01 = w_v[1, pl.ds(off, L)]
            w10 = w_v[2, pl.ds(off, L)]
            w11 = w_v[3, pl.ds(off, L)]

            def per_chan(c, carry):
                j00, j01, j10, j11, oidx = carry
                v00 = plsc.load_gather(feat_v, [j00])
                v01 = plsc.load_gather(feat_v, [j01])
                v10 = plsc.load_gather(feat_v, [j10])
                v11 = plsc.load_gather(feat_v, [j11])
                acc = v00 * w00 + v01 * w01 + v10 * w10 + v11 * w11
                out_v[0, pl.ds(oidx, L)] = acc
                return (j00 + HW, j01 + HW, j10 + HW, j11 + HW, oidx + P)

            lax.fori_loop(0, C, per_chan, (i00, i01, i10, i11, off),
                          unroll=2)
            return 0

        lax.fori_loop(0, n_pc, per_chunk, 0)

        pltpu.sync_copy(out_v, out_hbm.at[pl.ds(b, 1)])
        return 0

    lax.fori_loop(0, n_b, per_query, 0)


def kernel(image_index, grid, features):
    num_images, C, H, W = features.shape
    B, _, P = grid.shape
    CHW = C * H * W

    feat_flat = features.reshape(num_images * CHW)
    ii = image_index.astype(jnp.int32)

    body = functools.partial(_sc_body, num_images, C, H, W, B, P)
    f = pl.kernel(
        body,
        out_type=jax.ShapeDtypeStruct((B, C * P), jnp.float32),
        mesh=plsc.VectorSubcoreMesh(core_axis_name="c", subcore_axis_name="s",
                                    num_cores=NC, num_subcores=NS),
        compiler_params=pltpu.CompilerParams(needs_layout_passes=False),
        scratch_types=[
            pltpu.VMEM((B // NW,), jnp.int32),
            pltpu.VMEM((1, 2, P), jnp.float32),
            pltpu.VMEM((CHW,), jnp.float32),
            pltpu.VMEM((4, P), jnp.int32),
            pltpu.VMEM((4, P), jnp.float32),
            pltpu.VMEM((1, C * P), jnp.float32),
            pltpu.SemaphoreType.DMA,
        ],
    )
    out = f(ii, grid, feat_flat)
    return out.reshape(B, C, P)
